# baseline (speedup 1.0000x reference)
import functools

import jax
import jax.numpy as jnp
from jax import lax
from jax.experimental import pallas as pl
from jax.experimental.pallas import tpu as pltpu
from jax.experimental.pallas import tpu_sc as plsc

DIM = 1024
HID = 2048
E = 8
T = 2048
BT = 256
TILE = 256
CAP = T
NTILES = CAP // TILE


def _router_kernel(x_ref, rw_ref, xbf_ref, s1_ref, s2_ref, w1_ref, w2_ref,
                   cnt_ref, carry_ref):
    b = pl.program_id(0)

    @pl.when(b == 0)
    def _():
        carry_ref[...] = jnp.zeros_like(carry_ref)

    xb = x_ref[...]
    xbf_ref[...] = xb.astype(jnp.bfloat16)

    logits = jnp.dot(xb, rw_ref[...].T, preferred_element_type=jnp.float32)
    eidx = lax.broadcasted_iota(jnp.int32, logits.shape, 1)
    m1 = jnp.max(logits, axis=1, keepdims=True)
    i1 = jnp.min(jnp.where(logits == m1, eidx, E), axis=1, keepdims=True)
    masked = jnp.where(eidx == i1, -jnp.inf, logits)
    m2 = jnp.max(masked, axis=1, keepdims=True)
    i2 = jnp.min(jnp.where(masked == m2, eidx, E), axis=1, keepdims=True)
    denom = jnp.sum(jnp.exp(logits - m1), axis=1, keepdims=True)
    p1 = 1.0 / denom
    p2 = jnp.exp(m2 - m1) / denom
    s = p1 + p2 + 1e-8
    w1_ref[0] = (p1 / s)
    w2_ref[0] = (p2 / s)

    sel1 = (eidx == i1).astype(jnp.float32)
    sel2 = (eidx == i2).astype(jnp.float32)
    sel = sel1 + sel2
    ri = lax.broadcasted_iota(jnp.int32, (BT, BT), 0)
    ci = lax.broadcasted_iota(jnp.int32, (BT, BT), 1)
    ltri = (ri > ci).astype(jnp.bfloat16)
    pos = jnp.dot(ltri, sel.astype(jnp.bfloat16),
                  preferred_element_type=jnp.float32)
    pos = pos + carry_ref[...]
    base = (eidx * CAP).astype(jnp.float32)
    slotf = base + pos
    s1_ref[0] = jnp.sum(sel1 * slotf, axis=1, keepdims=True).astype(jnp.int32)
    s2_ref[0] = jnp.sum(sel2 * slotf, axis=1, keepdims=True).astype(jnp.int32)
    carry_ref[...] += jnp.sum(sel, axis=0, keepdims=True)

    @pl.when(b == pl.num_programs(0) - 1)
    def _():
        cnt_ref[...] = carry_ref[...].astype(jnp.int32)


def _run_router(x_flat, router_w):
    nb = T // BT
    return pl.pallas_call(
        _router_kernel,
        grid=(nb,),
        in_specs=[
            pl.BlockSpec((BT, DIM), lambda b: (b, 0)),
            pl.BlockSpec((E, DIM), lambda b: (0, 0)),
        ],
        out_specs=[
            pl.BlockSpec((BT, DIM), lambda b: (b, 0)),
            pl.BlockSpec((1, BT, 1), lambda b: (b, 0, 0)),
            pl.BlockSpec((1, BT, 1), lambda b: (b, 0, 0)),
            pl.BlockSpec((1, BT, 1), lambda b: (b, 0, 0)),
            pl.BlockSpec((1, BT, 1), lambda b: (b, 0, 0)),
            pl.BlockSpec((1, E), lambda b: (0, 0)),
        ],
        out_shape=[
            jax.ShapeDtypeStruct((T, DIM), jnp.bfloat16),
            jax.ShapeDtypeStruct((nb, BT, 1), jnp.int32),
            jax.ShapeDtypeStruct((nb, BT, 1), jnp.int32),
            jax.ShapeDtypeStruct((nb, BT, 1), jnp.float32),
            jax.ShapeDtypeStruct((nb, BT, 1), jnp.float32),
            jax.ShapeDtypeStruct((1, E), jnp.int32),
        ],
        scratch_shapes=[pltpu.VMEM((1, E), jnp.float32)],
        compiler_params=pltpu.CompilerParams(
            dimension_semantics=("arbitrary",),
        ),
    )(x_flat, router_w)


def _mlp_kernel(cnt_ref, xbf_ref, s1_ref, s2_ref, w1_ref, w2_ref,
                fc_ref, pj_ref, ys_ref, fcb_ref, pjb_ref):
    e = pl.program_id(0)
    t = pl.program_id(1)
    c_e = cnt_ref[e]

    @pl.when(jnp.logical_and(t == 0, c_e > 0))
    def _():
        fcb_ref[...] = fc_ref[0].astype(jnp.bfloat16)
        pjb_ref[...] = pj_ref[0].astype(jnp.bfloat16)

    @pl.when(t * TILE < c_e)
    def _():
        srow = (e * CAP + t * TILE
                + lax.broadcasted_iota(jnp.int32, (TILE, 1), 0))
        s1 = s1_ref[...]
        s2 = s2_ref[...]
        hit1 = s1 == srow
        hit2 = s2 == srow
        p = hit1.astype(jnp.bfloat16) + hit2.astype(jnp.bfloat16)
        xs = jnp.dot(p, xbf_ref[...], preferred_element_type=jnp.float32)
        wcol = jnp.sum(jnp.where(hit1, w1_ref[...], 0.0)
                       + jnp.where(hit2, w2_ref[...], 0.0),
                       axis=1, keepdims=True)
        h = jnp.dot(xs.astype(jnp.bfloat16), fcb_ref[...].T,
                    preferred_element_type=jnp.float32)
        h = jnp.square(jnp.maximum(h, 0.0))
        y = jnp.dot(h.astype(jnp.bfloat16), pjb_ref[...].T,
                    preferred_element_type=jnp.float32)
        ys_ref[...] = y * wcol


def _run_mlp(counts, x_bf, s1, s2, w1, w2, fc_w, proj_w):
    def ys_map(e, t, cnt):
        ntile = jnp.maximum(lax.div(cnt[e] + (TILE - 1), TILE), 1)
        return (e * NTILES + jnp.minimum(t, ntile - 1), 0)

    grid_spec = pltpu.PrefetchScalarGridSpec(
        num_scalar_prefetch=1,
        grid=(E, NTILES),
        in_specs=[
            pl.BlockSpec((T, DIM), lambda e, t, cnt: (0, 0)),
            pl.BlockSpec((1, T), lambda e, t, cnt: (0, 0)),
            pl.BlockSpec((1, T), lambda e, t, cnt: (0, 0)),
            pl.BlockSpec((1, T), lambda e, t, cnt: (0, 0)),
            pl.BlockSpec((1, T), lambda e, t, cnt: (0, 0)),
            pl.BlockSpec((1, HID, DIM), lambda e, t, cnt: (e, 0, 0)),
            pl.BlockSpec((1, DIM, HID), lambda e, t, cnt: (e, 0, 0)),
        ],
        out_specs=pl.BlockSpec((TILE, DIM), ys_map),
        scratch_shapes=[
            pltpu.VMEM((HID, DIM), jnp.bfloat16),
            pltpu.VMEM((DIM, HID), jnp.bfloat16),
        ],
    )
    return pl.pallas_call(
        _mlp_kernel,
        grid_spec=grid_spec,
        out_shape=jax.ShapeDtypeStruct((E * CAP, DIM), jnp.float32),
        compiler_params=pltpu.CompilerParams(
            dimension_semantics=("arbitrary", "arbitrary"),
        ),
    )(counts, x_bf, s1, s2, w1, w2, fc_w, proj_w)


_SC_CHUNK = 32


def _run_combine(ys, s1, s2):
    mesh = plsc.VectorSubcoreMesh(core_axis_name="c", subcore_axis_name="s")
    n_workers = 32
    per_w = T // n_workers

    @functools.partial(
        pl.kernel,
        mesh=mesh,
        out_type=jax.ShapeDtypeStruct((T, DIM), jnp.float32),
        scratch_types=[
            pltpu.VMEM((_SC_CHUNK,), jnp.int32),
            pltpu.VMEM((_SC_CHUNK,), jnp.int32),
            pltpu.VMEM((_SC_CHUNK, DIM), jnp.float32),
            pltpu.VMEM((_SC_CHUNK, DIM), jnp.float32),
            pltpu.SemaphoreType.DMA,
        ],
    )
    def combine(ys_hbm, s1_hbm, s2_hbm, out_hbm, i1_v, i2_v, ra_v, rb_v, sem):
        wid = lax.axis_index("s") * 2 + lax.axis_index("c")
        base = wid * per_w
        for chunk in range(per_w // _SC_CHUNK):
            off = base + chunk * _SC_CHUNK
            pltpu.sync_copy(s1_hbm.at[pl.ds(off, _SC_CHUNK)], i1_v)
            pltpu.sync_copy(s2_hbm.at[pl.ds(off, _SC_CHUNK)], i2_v)
            ca = pltpu.async_copy(ys_hbm.at[i1_v], ra_v, sem)
            cb = pltpu.async_copy(ys_hbm.at[i2_v], rb_v, sem)
            ca.wait()
            cb.wait()

            @pl.loop(0, _SC_CHUNK)
            def _(r):
                @pl.loop(0, DIM // 16)
                def _(c):
                    sl = (r, pl.ds(c * 16, 16))
                    ra_v.at[*sl][...] = ra_v.at[*sl][...] + rb_v.at[*sl][...]

            pltpu.sync_copy(ra_v, out_hbm.at[pl.ds(off, _SC_CHUNK)])

    return combine(ys, s1, s2)


def kernel(x, router_w, fc_w, proj_w):
    bsz, seq_len, dim = x.shape
    x_flat = x.reshape(-1, dim)
    x_bf, s1, s2, w1, w2, counts = _run_router(x_flat, router_w)
    s1f = s1.reshape(1, T)
    s2f = s2.reshape(1, T)
    w1f = w1.reshape(1, T)
    w2f = w2.reshape(1, T)
    ys = _run_mlp(counts.reshape(E), x_bf, s1f, s2f, w1f, w2f, fc_w, proj_w)
    out = _run_combine(ys, s1.reshape(T), s2.reshape(T))
    return out.reshape(bsz, seq_len, dim), jnp.float32(0.0)
